# Initial kernel scaffold; baseline (speedup 1.0000x reference)
import jax
import jax.numpy as jnp
from jax.experimental import pallas as pl


def kernel(indices, table_0, table_1):
    raise NotImplementedError("write your pallas kernel here")


# speedup vs baseline: 1.6223x; 1.6223x over previous
import functools

import jax
import jax.numpy as jnp
from jax import lax
from jax.experimental import pallas as pl
from jax.experimental.pallas import tpu as pltpu
from jax.experimental.pallas import tpu_sc as plsc

F = 2
B = 16384
H = 20
D = 32
N = B * H
NC = 2
NS = 16
NW = NC * NS
PER_W = N // NW
CH = 1024
NCH = PER_W // CH
NBUF = 2

_mesh = plsc.VectorSubcoreMesh(core_axis_name="c", subcore_axis_name="s")


@functools.partial(
    pl.kernel,
    mesh=_mesh,
    compiler_params=pltpu.CompilerParams(use_tc_tiling_on_sc=False),
    out_type=jax.ShapeDtypeStruct((F, N, D), jnp.float32),
    scratch_types=[
        pltpu.VMEM((NBUF, CH), jnp.int32),
        pltpu.VMEM((NBUF, CH, D), jnp.float32),
        pltpu.SemaphoreType.DMA,
        pltpu.SemaphoreType.DMA,
    ],
)
def _lookup(idx_hbm, t0_hbm, t1_hbm, out_hbm, idx_v, rows_v, sem0, sem1):
    wid = lax.axis_index("s") * NC + lax.axis_index("c")
    base = wid * PER_W
    tables = (t0_hbm, t1_hbm)
    sems = (sem0, sem1)
    chunks = [(f, j) for f in range(F) for j in range(NCH)]

    def start(slot, f, j):
        off = base + j * CH
        pltpu.sync_copy(idx_hbm.at[f, pl.ds(off, CH)], idx_v.at[slot])
        return pltpu.async_copy(
            tables[f].at[idx_v.at[slot]], rows_v.at[slot], sems[slot])

    inflight = {0: start(0, *chunks[0])}
    for i, (f, j) in enumerate(chunks):
        slot = i % NBUF
        if i + 1 < len(chunks):
            nslot = (i + 1) % NBUF
            inflight[nslot] = start(nslot, *chunks[i + 1])
        inflight[slot].wait()
        off = base + j * CH
        pltpu.sync_copy(rows_v.at[slot], out_hbm.at[f, pl.ds(off, CH)])


def kernel(indices, table_0, table_1):
    idx = indices.reshape(F, N).astype(jnp.int32)
    out = _lookup(idx, table_0, table_1)
    return out.reshape(F, B, H, D)
